# baseline (speedup 1.0000x reference)
import functools

import jax
import jax.numpy as jnp
from jax import lax
from jax.experimental import pallas as pl
from jax.experimental.pallas import tpu as pltpu
from jax.experimental.pallas import tpu_sc as plsc

_VOCAB = 100000
_MAX_POS = 2048
_D = 768
_B = 4
_S = 2048
_TOK = _B * _S
_NC = 2
_NS = 16
_NW = _NC * _NS
_SPAN = _S // _NW
_C = 32
_HALVES = _SPAN // _C
_NCH = _B * _HALVES
_LANES = 16
_RING = 4


def _emb_body(ids_hbm, wte_hbm, wpe_hbm, out_hbm,
              idx_all, a0, a1, a2, a3, p_v,
              gsem0, gsem1, gsem2, gsem3,
              osem0, osem1, osem2, osem3, isem, psem):
    wid = lax.axis_index("s") * _NC + lax.axis_index("c")
    pos0 = wid * _SPAN
    a = (a0, a1, a2, a3)
    gsem = (gsem0, gsem1, gsem2, gsem3)
    osem = (osem0, osem1, osem2, osem3)

    def tok_off(k):
        h, b = divmod(k, _B)
        return b * _S + pos0 + h * _C

    idescs = [
        pltpu.async_copy(ids_hbm.at[pl.ds(tok_off(k), _C)], idx_all.at[k], isem)
        for k in range(_NCH)
    ]
    pd = pltpu.async_copy(wpe_hbm.at[pl.ds(pos0, _C)], p_v, psem)

    gd = [None] * _RING
    for k in range(_RING):
        idescs[k].wait()
        gd[k] = pltpu.async_copy(wte_hbm.at[idx_all.at[k]], a[k], gsem[k])
    od = [None] * _RING

    for k in range(_NCH):
        s = k % _RING
        if k == 0 or k == _B:
            pd.wait()
        gd[s].wait()
        if 2 <= k and k + 2 < _NCH:
            s2 = (k + 2) % _RING
            idescs[k + 2].wait()
            od[s2].wait()
            od[s2] = None
            gd[s2] = pltpu.async_copy(wte_hbm.at[idx_all.at[k + 2]], a[s2], gsem[s2])
        a_s = a[s]

        @pl.loop(0, _C, unroll=4)
        def _row(r, a_s=a_s):
            for c in range(_D // _LANES):
                sl = pl.ds(c * _LANES, _LANES)
                plsc.addupdate(a_s.at[r, sl], p_v[r, sl])

        od[s] = pltpu.async_copy(a_s, out_hbm.at[pl.ds(tok_off(k), _C)], osem[s])
        if k == _B - 1:
            pd = pltpu.async_copy(wpe_hbm.at[pl.ds(pos0 + _C, _C)], p_v, psem)

    for s in range(_RING):
        if od[s] is not None:
            od[s].wait()


@functools.partial(
    pl.kernel,
    out_type=jax.ShapeDtypeStruct((_TOK, _D), jnp.float32),
    mesh=plsc.VectorSubcoreMesh(
        core_axis_name="c", subcore_axis_name="s",
        num_cores=_NC, num_subcores=_NS,
    ),
    scratch_types=[
        pltpu.VMEM((_NCH, _C), jnp.int32),
        pltpu.VMEM((_C, _D), jnp.float32),
        pltpu.VMEM((_C, _D), jnp.float32),
        pltpu.VMEM((_C, _D), jnp.float32),
        pltpu.VMEM((_C, _D), jnp.float32),
        pltpu.VMEM((_C, _D), jnp.float32),
        pltpu.SemaphoreType.DMA,
        pltpu.SemaphoreType.DMA,
        pltpu.SemaphoreType.DMA,
        pltpu.SemaphoreType.DMA,
        pltpu.SemaphoreType.DMA,
        pltpu.SemaphoreType.DMA,
        pltpu.SemaphoreType.DMA,
        pltpu.SemaphoreType.DMA,
        pltpu.SemaphoreType.DMA,
        pltpu.SemaphoreType.DMA,
    ],
)
def _emb_lookup(ids_hbm, wte_hbm, wpe_hbm, out_hbm,
                idx_all, a0, a1, a2, a3, p_v,
                gsem0, gsem1, gsem2, gsem3,
                osem0, osem1, osem2, osem3, isem, psem):
    _emb_body(ids_hbm, wte_hbm, wpe_hbm, out_hbm,
              idx_all, a0, a1, a2, a3, p_v,
              gsem0, gsem1, gsem2, gsem3,
              osem0, osem1, osem2, osem3, isem, psem)


def kernel(input_ids, attention_mask, hidden_states, wte, wpe):
    input_shape = input_ids.shape
    input_ids = input_ids.reshape(-1, input_shape[-1])
    ids_flat = input_ids.reshape(-1)
    hs = _emb_lookup(ids_flat, wte, wpe)
    hs = hs.reshape(input_ids.shape[0], input_ids.shape[1], _D)
    return (input_ids, attention_mask, hs)


# speedup vs baseline: 1.1075x; 1.1075x over previous
import functools

import jax
import jax.numpy as jnp
from jax import lax
from jax.experimental import pallas as pl
from jax.experimental.pallas import tpu as pltpu
from jax.experimental.pallas import tpu_sc as plsc

_VOCAB = 100000
_MAX_POS = 2048
_D = 768
_B = 4
_S = 2048
_TOK = _B * _S
_NC = 2
_NS = 16
_NW = _NC * _NS
_SPAN = _S // _NW
_C = 32
_HALVES = _SPAN // _C
_NCH = _B * _HALVES
_LANES = 16
_RING = 4


def _emb_body(ids_hbm, wte_hbm, wpe_hbm, out_hbm,
              idx_all, a0, a1, a2, a3, p_v,
              gsem0, gsem1, gsem2, gsem3,
              osem0, osem1, osem2, osem3, isem, psem):
    wid = lax.axis_index("s") * _NC + lax.axis_index("c")
    pos0 = wid * _SPAN
    a = (a0, a1, a2, a3)
    gsem = (gsem0, gsem1, gsem2, gsem3)
    osem = (osem0, osem1, osem2, osem3)

    def tok_off(k):
        h, b = divmod(k, _B)
        return b * _S + pos0 + h * _C

    idescs = [
        pltpu.async_copy(ids_hbm.at[pl.ds(tok_off(k), _C)], idx_all.at[k], isem)
        for k in range(_NCH)
    ]
    pd = pltpu.async_copy(wpe_hbm.at[pl.ds(pos0, _C)], p_v, psem)

    gd = [None] * _RING
    for k in range(_RING):
        idescs[k].wait()
        gd[k] = pltpu.async_copy(wte_hbm.at[idx_all.at[k]], a[k], gsem[k])
    od = [None] * _RING

    for k in range(_NCH):
        s = k % _RING
        if k == 0 or k == _B:
            pd.wait()
        gd[s].wait()
        if 2 <= k and k + 2 < _NCH:
            s2 = (k + 2) % _RING
            idescs[k + 2].wait()
            od[s2].wait()
            od[s2] = None
            gd[s2] = pltpu.async_copy(wte_hbm.at[idx_all.at[k + 2]], a[s2], gsem[s2])
        a_s = a[s]

        @pl.loop(0, _C)
        def _row(r, a_s=a_s):
            for c in range(_D // _LANES):
                sl = pl.ds(c * _LANES, _LANES)
                plsc.addupdate(a_s.at[r, sl], p_v[r, sl])

        od[s] = pltpu.async_copy(a_s, out_hbm.at[pl.ds(tok_off(k), _C)], osem[s])
        if k == _B - 1:
            pd = pltpu.async_copy(wpe_hbm.at[pl.ds(pos0 + _C, _C)], p_v, psem)

    for s in range(_RING):
        if od[s] is not None:
            od[s].wait()


@functools.partial(
    pl.kernel,
    out_type=jax.ShapeDtypeStruct((_TOK, _D), jnp.float32),
    mesh=plsc.VectorSubcoreMesh(
        core_axis_name="c", subcore_axis_name="s",
        num_cores=_NC, num_subcores=_NS,
    ),
    scratch_types=[
        pltpu.VMEM((_NCH, _C), jnp.int32),
        pltpu.VMEM((_C, _D), jnp.float32),
        pltpu.VMEM((_C, _D), jnp.float32),
        pltpu.VMEM((_C, _D), jnp.float32),
        pltpu.VMEM((_C, _D), jnp.float32),
        pltpu.VMEM((_C, _D), jnp.float32),
        pltpu.SemaphoreType.DMA,
        pltpu.SemaphoreType.DMA,
        pltpu.SemaphoreType.DMA,
        pltpu.SemaphoreType.DMA,
        pltpu.SemaphoreType.DMA,
        pltpu.SemaphoreType.DMA,
        pltpu.SemaphoreType.DMA,
        pltpu.SemaphoreType.DMA,
        pltpu.SemaphoreType.DMA,
        pltpu.SemaphoreType.DMA,
    ],
)
def _emb_lookup(ids_hbm, wte_hbm, wpe_hbm, out_hbm,
                idx_all, a0, a1, a2, a3, p_v,
                gsem0, gsem1, gsem2, gsem3,
                osem0, osem1, osem2, osem3, isem, psem):
    _emb_body(ids_hbm, wte_hbm, wpe_hbm, out_hbm,
              idx_all, a0, a1, a2, a3, p_v,
              gsem0, gsem1, gsem2, gsem3,
              osem0, osem1, osem2, osem3, isem, psem)


def kernel(input_ids, attention_mask, hidden_states, wte, wpe):
    input_shape = input_ids.shape
    input_ids = input_ids.reshape(-1, input_shape[-1])
    ids_flat = input_ids.reshape(-1)
    hs = _emb_lookup(ids_flat, wte, wpe)
    hs = hs.reshape(input_ids.shape[0], input_ids.shape[1], _D)
    return (input_ids, attention_mask, hs)
